# baseline (speedup 1.0000x reference)
import functools

import jax
import jax.numpy as jnp
from jax import lax
from jax.experimental import pallas as pl
from jax.experimental.pallas import tpu as pltpu
from jax.experimental.pallas import tpu_sc as plsc

D = 64
NC = 2
NS = 16
NW = NC * NS
CH = 128
COLS = 65536


def _proj_body(tT_ref, w_ref, b_ref, o0_ref, o1_ref):
    x = tT_ref[...]
    w = w_ref[...]
    p = jnp.dot(w, x, preferred_element_type=jnp.float32)
    p = p + b_ref[...]
    o0_ref[...] = p[0]
    o1_ref[...] = p[1]


@functools.partial(jax.jit, static_argnums=(0, 1))
def _router(B, V, idsT, token_embT, fc_w, fc_b2):
    ncols = (V + COLS - 1) // COLS
    p0, p1 = pl.pallas_call(
        _proj_body,
        grid=(ncols,),
        in_specs=[
            pl.BlockSpec((D, COLS), lambda c: (0, c)),
            pl.BlockSpec((2, D), lambda c: (0, 0)),
            pl.BlockSpec((2, 1), lambda c: (0, 0)),
        ],
        out_specs=[
            pl.BlockSpec((COLS,), lambda c: (c,)),
            pl.BlockSpec((COLS,), lambda c: (c,)),
        ],
        out_shape=[
            jax.ShapeDtypeStruct((V,), jnp.float32),
            jax.ShapeDtypeStruct((V,), jnp.float32),
        ],
    )(token_embT, fc_w, fc_b2)

    bpw = B // NW
    nchunk = bpw // CH
    mesh = plsc.VectorSubcoreMesh(core_axis_name="c", subcore_axis_name="s")

    @functools.partial(
        pl.kernel,
        mesh=mesh,
        compiler_params=pltpu.CompilerParams(needs_layout_passes=False),
        out_type=jax.ShapeDtypeStruct((2, B), jnp.float32),
        scratch_types=[
            pltpu.VMEM((bpw,), jnp.int32),
            pltpu.VMEM((bpw,), jnp.float32),
            pltpu.VMEM((bpw,), jnp.float32),
            pltpu.SemaphoreType.DMA,
        ],
    )
    def gather_k(p0_hbm, p1_hbm, idsT_hbm, out_hbm, idsv, g0, g1, sem):
        wid = lax.axis_index("s") * NC + lax.axis_index("c")
        base = wid * bpw

        pltpu.sync_copy(idsT_hbm.at[0, pl.ds(base, bpw)], idsv)

        copies = []
        for j in range(nchunk):
            copies.append(pltpu.async_copy(
                p0_hbm.at[idsv.at[pl.ds(j * CH, CH)]],
                g0.at[pl.ds(j * CH, CH)], sem))
            copies.append(pltpu.async_copy(
                p1_hbm.at[idsv.at[pl.ds(j * CH, CH)]],
                g1.at[pl.ds(j * CH, CH)], sem))
        for c in copies:
            c.wait()

        pltpu.sync_copy(g0, out_hbm.at[0, pl.ds(base, bpw)])
        pltpu.sync_copy(g1, out_hbm.at[1, pl.ds(base, bpw)])

    return gather_k(p0, p1, idsT)


def kernel(ids, token_emb, fc_w, fc_b):
    B = ids.shape[0]
    V = token_emb.shape[0]
    idsT = ids.astype(jnp.int32).T
    token_embT = token_emb.T
    out2 = _router(B, V, idsT, token_embT, fc_w, fc_b.reshape(2, 1))
    return out2.T
